# baseline (speedup 1.0000x reference)
import functools

import jax
import jax.numpy as jnp
from jax.experimental import pallas as pl
from jax.experimental.pallas import tpu as pltpu

K = 64
N = 10000
N_PAD = 10240
ROWS, COLS = 8, 1280


NV = COLS // 128


def _tree(op, xs):
    while len(xs) > 1:
        xs = [op(xs[2 * i], xs[2 * i + 1]) for i in range(len(xs) // 2)] + (
            [xs[-1]] if len(xs) % 2 else [])
    return xs[0]


def _topk_kernel(w_ref, idx_ref, val_ref):
    r = jax.lax.broadcasted_iota(jnp.int32, (ROWS, COLS), 0)
    c = jax.lax.broadcasted_iota(jnp.int32, (ROWS, COLS), 1)
    flat = r * COLS + c
    a = jnp.abs(w_ref[...])
    a = jnp.where(flat < N, a, -1.0)
    flats = [flat[:, v * 128:(v + 1) * 128] for v in range(NV)]
    lane = jax.lax.broadcasted_iota(jnp.int32, (1, K), 1)

    BIG = jnp.int32(2**30)

    def _gmax(x):
        parts = [x[:, v * 128:(v + 1) * 128] for v in range(NV)]
        vm = _tree(jnp.maximum, parts)
        return jnp.max(jnp.max(vm, axis=0, keepdims=True),
                       axis=1, keepdims=True)

    def _gmin_i(x):
        parts = [x[:, v * 128:(v + 1) * 128] for v in range(NV)]
        vm = _tree(jnp.minimum, parts)
        return jnp.min(jnp.min(vm, axis=0, keepdims=True),
                       axis=1, keepdims=True)

    def _gcnt(e):
        return jnp.sum(jnp.sum(e.astype(jnp.int32), axis=0, keepdims=True),
                       axis=1, keepdims=True)

    def body(i, carry):
        a, idxv, valv = carry
        m1 = _gmax(a)
        j1 = _gmin_i(jnp.where(a == m1, flat, BIG))
        cnt1 = _gcnt(a == m1)
        m2b = _gmax(jnp.where(a >= m1, -1.0, a))
        m2 = jnp.where(cnt1 >= 2, m1, m2b)

        j2 = _gmin_i(jnp.where((a == m2) & (flat != j1), flat, BIG))
        cnt2 = _gcnt(a == m2)
        m3b = _gmax(jnp.where(a >= m2, -1.0, a))
        ext2 = jnp.where(m2 == m1, 2, 1)
        m3 = jnp.where(cnt2 > ext2, m2, m3b)

        j3 = _gmin_i(jnp.where((a == m3) & (flat != j1) & (flat != j2),
                               flat, BIG))
        cnt3 = _gcnt(a == m3)
        m4b = _gmax(jnp.where(a >= m3, -1.0, a))
        ext3 = jnp.where(m3 == m2, ext2 + 1, 1)
        m4 = jnp.where(cnt3 > ext3, m3, m4b)

        j4 = _gmin_i(jnp.where((a == m4) & (flat != j1) & (flat != j2)
                               & (flat != j3), flat, BIG))

        idxv = jnp.where(lane == 4 * i, j1, idxv)
        idxv = jnp.where(lane == 4 * i + 1, j2, idxv)
        idxv = jnp.where(lane == 4 * i + 2, j3, idxv)
        idxv = jnp.where(lane == 4 * i + 3, j4, idxv)
        valv = jnp.where(lane == 4 * i, m1, valv)
        valv = jnp.where(lane == 4 * i + 1, m2, valv)
        valv = jnp.where(lane == 4 * i + 2, m3, valv)
        valv = jnp.where(lane == 4 * i + 3, m4, valv)
        a = jnp.where((flat == j1) | (flat == j2) | (flat == j3)
                      | (flat == j4), -1.0, a)
        return a, idxv, valv

    idx0 = jnp.zeros((1, K), jnp.int32)
    val0 = jnp.zeros((1, K), jnp.float32)
    _, idxv, valv = jax.lax.fori_loop(0, K // 4, body, (a, idx0, val0))
    idx_ref[...] = idxv
    val_ref[...] = valv


def _run_topk(W):
    wp = jnp.pad(W, (0, N_PAD - N)).reshape(ROWS, COLS)
    idx2d, val2d = pl.pallas_call(
        _topk_kernel,
        out_shape=(
            jax.ShapeDtypeStruct((1, K), jnp.int32),
            jax.ShapeDtypeStruct((1, K), jnp.float32),
        ),
    )(wp)
    return idx2d.reshape(K), val2d.reshape(K)


G = 32
STEPS = K // G


def _mlp_kernel(idx_ref, val_ref, *refs):
    x_refs = refs[:G]
    w1_ref, b1_ref, gamma_ref, beta_ref, w2_ref, b2_ref, out_ref, acc_ref = refs[G:]
    g = pl.program_id(0)

    @pl.when(g == 0)
    def _():
        acc_ref[...] = jnp.zeros_like(acc_ref)

    xcat = jnp.concatenate(
        [x_refs[j][:, 0, 0, :] * val_ref[g * G + j] for j in range(G)],
        axis=1)
    acc_ref[...] += jax.lax.dot_general(
        xcat, w1_ref[...], (((1,), (1,)), ((), ())),
        preferred_element_type=jnp.float32)

    @pl.when(g == STEPS - 1)
    def _():
        mp = acc_ref[...] + b1_ref[...]
        mean = jnp.mean(mp, axis=0, keepdims=True)
        var = jnp.mean((mp - mean) ** 2, axis=0, keepdims=True)
        mp = (mp - mean) * jax.lax.rsqrt(var + 1e-5)
        mp = mp * gamma_ref[...] + beta_ref[...]
        mp = jnp.maximum(mp, 0.0)
        out_ref[...] = jax.lax.dot_general(
            mp, w2_ref[...], (((1,), (1,)), ((), ())),
            preferred_element_type=jnp.float32) + b2_ref[...]


def kernel(x, W, W1, b1, gamma, beta, W2, b2):
    B, _, F = x.shape
    H = W1.shape[0]
    O = W2.shape[0]
    topk_idx, topk_val = _run_topk(W)

    x4 = x.reshape(B, N, 1, F)
    out = pl.pallas_call(
        _mlp_kernel,
        grid_spec=pltpu.PrefetchScalarGridSpec(
            num_scalar_prefetch=2,
            grid=(STEPS,),
            in_specs=[
                pl.BlockSpec((B, 1, 1, F),
                             functools.partial(
                                 lambda j, g, i, v: (0, i[g * G + j], 0, 0), j))
                for j in range(G)
            ] + [
                pl.BlockSpec((H, G * F), lambda g, i, v: (0, g)),
                pl.BlockSpec((1, H), lambda g, i, v: (0, 0)),
                pl.BlockSpec((1, H), lambda g, i, v: (0, 0)),
                pl.BlockSpec((1, H), lambda g, i, v: (0, 0)),
                pl.BlockSpec((O, H), lambda g, i, v: (0, 0)),
                pl.BlockSpec((1, O), lambda g, i, v: (0, 0)),
            ],
            out_specs=pl.BlockSpec((B, O), lambda g, i, v: (0, 0)),
            scratch_shapes=[pltpu.VMEM((B, H), jnp.float32)],
        ),
        out_shape=jax.ShapeDtypeStruct((B, O), jnp.float32),
    )(topk_idx, topk_val, *([x4] * G), W1, b1.reshape(1, H),
      gamma.reshape(1, H), beta.reshape(1, H), W2, b2.reshape(1, O))
    return (out, topk_idx)


# speedup vs baseline: 1.0018x; 1.0018x over previous
import functools

import jax
import jax.numpy as jnp
from jax.experimental import pallas as pl
from jax.experimental.pallas import tpu as pltpu

K = 64
N = 10000
N_PAD = 10240
ROWS, COLS = 8, 1280


NV = COLS // 128


def _tree(op, xs):
    while len(xs) > 1:
        xs = [op(xs[2 * i], xs[2 * i + 1]) for i in range(len(xs) // 2)] + (
            [xs[-1]] if len(xs) % 2 else [])
    return xs[0]


def _topk_kernel(w_ref, idx_ref, val_ref):
    r = jax.lax.broadcasted_iota(jnp.int32, (ROWS, COLS), 0)
    c = jax.lax.broadcasted_iota(jnp.int32, (ROWS, COLS), 1)
    flat = r * COLS + c
    a = jnp.abs(w_ref[...])
    a = jnp.where(flat < N, a, -1.0)
    flats = [flat[:, v * 128:(v + 1) * 128] for v in range(NV)]
    lane = jax.lax.broadcasted_iota(jnp.int32, (1, K), 1)

    BIG = jnp.int32(2**30)

    def _gmax(x):
        parts = [x[:, v * 128:(v + 1) * 128] for v in range(NV)]
        vm = _tree(jnp.maximum, parts)
        return jnp.max(jnp.max(vm, axis=0, keepdims=True),
                       axis=1, keepdims=True)

    def _gmin_i(x):
        parts = [x[:, v * 128:(v + 1) * 128] for v in range(NV)]
        vm = _tree(jnp.minimum, parts)
        return jnp.min(jnp.min(vm, axis=0, keepdims=True),
                       axis=1, keepdims=True)

    def _gcnt(e):
        return jnp.sum(jnp.sum(e.astype(jnp.int32), axis=0, keepdims=True),
                       axis=1, keepdims=True)

    T_EXT = 8

    def body(i, carry):
        a, idxv, valv = carry
        m = _gmax(a)
        taken = jnp.zeros((1, 1), jnp.int32)
        ex = jnp.zeros(a.shape, jnp.bool_)
        for t in range(T_EXT):
            j = _gmin_i(jnp.where((a == m) & ~ex, flat, BIG))
            idxv = jnp.where(lane == T_EXT * i + t, j, idxv)
            valv = jnp.where(lane == T_EXT * i + t, m, valv)
            ex = ex | (flat == j)
            if t < T_EXT - 1:
                cnt = _gcnt(a == m)
                mb = _gmax(jnp.where(a >= m, -1.0, a))
                taken = taken + 1
                cond = cnt > taken
                m = jnp.where(cond, m, mb)
                taken = jnp.where(cond, taken, 0)
        a = jnp.where(ex, -1.0, a)
        return a, idxv, valv

    idx0 = jnp.zeros((1, K), jnp.int32)
    val0 = jnp.zeros((1, K), jnp.float32)
    _, idxv, valv = jax.lax.fori_loop(0, K // T_EXT, body, (a, idx0, val0))
    idx_ref[...] = idxv
    val_ref[...] = valv


def _run_topk(W):
    wp = jnp.pad(W, (0, N_PAD - N)).reshape(ROWS, COLS)
    idx2d, val2d = pl.pallas_call(
        _topk_kernel,
        out_shape=(
            jax.ShapeDtypeStruct((1, K), jnp.int32),
            jax.ShapeDtypeStruct((1, K), jnp.float32),
        ),
    )(wp)
    return idx2d.reshape(K), val2d.reshape(K)


G = 16
STEPS = K // G


def _mlp_kernel(idx_ref, val_ref, *refs):
    x_refs = refs[:G]
    w1_ref, b1_ref, gamma_ref, beta_ref, w2_ref, b2_ref, out_ref, acc_ref = refs[G:]
    g = pl.program_id(0)

    @pl.when(g == 0)
    def _():
        acc_ref[...] = jnp.zeros_like(acc_ref)

    xcat = jnp.concatenate(
        [x_refs[j][:, 0, 0, :] * val_ref[g * G + j] for j in range(G)],
        axis=1)
    acc_ref[...] += jax.lax.dot_general(
        xcat, w1_ref[...], (((1,), (1,)), ((), ())),
        preferred_element_type=jnp.float32)

    @pl.when(g == STEPS - 1)
    def _():
        mp = acc_ref[...] + b1_ref[...]
        mean = jnp.mean(mp, axis=0, keepdims=True)
        var = jnp.mean((mp - mean) ** 2, axis=0, keepdims=True)
        mp = (mp - mean) * jax.lax.rsqrt(var + 1e-5)
        mp = mp * gamma_ref[...] + beta_ref[...]
        mp = jnp.maximum(mp, 0.0)
        out_ref[...] = jax.lax.dot_general(
            mp, w2_ref[...], (((1,), (1,)), ((), ())),
            preferred_element_type=jnp.float32) + b2_ref[...]


def kernel(x, W, W1, b1, gamma, beta, W2, b2):
    B, _, F = x.shape
    H = W1.shape[0]
    O = W2.shape[0]
    topk_idx, topk_val = _run_topk(W)

    x4 = x.reshape(B, N, 1, F)
    out = pl.pallas_call(
        _mlp_kernel,
        grid_spec=pltpu.PrefetchScalarGridSpec(
            num_scalar_prefetch=2,
            grid=(STEPS,),
            in_specs=[
                pl.BlockSpec((B, 1, 1, F),
                             functools.partial(
                                 lambda j, g, i, v: (0, i[g * G + j], 0, 0), j))
                for j in range(G)
            ] + [
                pl.BlockSpec((H, G * F), lambda g, i, v: (0, g)),
                pl.BlockSpec((1, H), lambda g, i, v: (0, 0)),
                pl.BlockSpec((1, H), lambda g, i, v: (0, 0)),
                pl.BlockSpec((1, H), lambda g, i, v: (0, 0)),
                pl.BlockSpec((O, H), lambda g, i, v: (0, 0)),
                pl.BlockSpec((1, O), lambda g, i, v: (0, 0)),
            ],
            out_specs=pl.BlockSpec((B, O), lambda g, i, v: (0, 0)),
            scratch_shapes=[pltpu.VMEM((B, H), jnp.float32)],
        ),
        out_shape=jax.ShapeDtypeStruct((B, O), jnp.float32),
    )(topk_idx, topk_val, *([x4] * G), W1, b1.reshape(1, H),
      gamma.reshape(1, H), beta.reshape(1, H), W2, b2.reshape(1, O))
    return (out, topk_idx)
